# Initial kernel scaffold; baseline (speedup 1.0000x reference)
import functools

import jax
import jax.numpy as jnp
from jax import lax
from jax.experimental import pallas as pl
from jax.experimental.pallas import tpu as pltpu
from jax.experimental.pallas import tpu_sc as plsc

NC = 2
NS = 16
NW = NC * NS
CHUNK = 128
LANES = 16


def _sc_aggregate(n_nodes, n_pad, e_pad, d, with_deg):
  e_per_tile = e_pad // NW
  n_chunks = e_per_tile // CHUNK
  rows_per_tile = n_pad // NS

  out_type = [jax.ShapeDtypeStruct((NC, n_pad, d), jnp.float32)]
  scratch = [
      pltpu.VMEM((CHUNK,), jnp.int32),
      pltpu.VMEM((CHUNK,), jnp.int32),
      pltpu.VMEM((CHUNK, d), jnp.float32),
      pltpu.VMEM_SHARED((n_pad, d), jnp.float32),
      pltpu.SemaphoreType.DMA,
  ]
  if with_deg:
    out_type.append(jax.ShapeDtypeStruct((NC, n_pad, LANES), jnp.float32))
    scratch += [
        pltpu.VMEM((CHUNK, LANES), jnp.float32),
        pltpu.VMEM((CHUNK, LANES), jnp.float32),
        pltpu.VMEM_SHARED((n_pad, LANES), jnp.float32),
    ]

  mesh = plsc.VectorSubcoreMesh(
      core_axis_name="c", subcore_axis_name="s", num_cores=NC,
      num_subcores=NS)

  def body(y_hbm, src_hbm, dst_hbm, *refs):
    if with_deg:
      (agg_out, deg_out, src_v, dst_v, rows_v, agg_sh, sem,
       ones_v, z16_v, deg_sh) = refs
    else:
      agg_out, src_v, dst_v, rows_v, agg_sh, sem = refs

    cid = lax.axis_index("c")
    sid = lax.axis_index("s")

    def zrow(i, _):
      def zcol(j, _):
        rows_v[i, pl.ds(j * LANES, LANES)] = jnp.zeros((LANES,), jnp.float32)
        return 0
      return lax.fori_loop(0, d // LANES, zcol, 0)
    lax.fori_loop(0, CHUNK, zrow, 0)

    r0 = sid * rows_per_tile
    n_zcopies = rows_per_tile // CHUNK
    for k in range(n_zcopies):
      pltpu.sync_copy(rows_v, agg_sh.at[pl.ds(r0 + k * CHUNK, CHUNK)])
    rem = rows_per_tile - n_zcopies * CHUNK
    if rem:
      pltpu.sync_copy(rows_v.at[pl.ds(0, rem)],
                      agg_sh.at[pl.ds(r0 + n_zcopies * CHUNK, rem)])

    if with_deg:
      def orow(i, _):
        ones_v[i, :] = jnp.ones((LANES,), jnp.float32)
        z16_v[i, :] = jnp.zeros((LANES,), jnp.float32)
        return 0
      lax.fori_loop(0, CHUNK, orow, 0)
      for k in range(n_zcopies):
        pltpu.sync_copy(z16_v, deg_sh.at[pl.ds(r0 + k * CHUNK, CHUNK)])
      if rem:
        pltpu.sync_copy(z16_v.at[pl.ds(0, rem)],
                        deg_sh.at[pl.ds(r0 + n_zcopies * CHUNK, rem)])

    plsc.subcore_barrier()

    wid = cid * NS + sid
    e0 = pl.multiple_of(wid * e_per_tile, CHUNK)

    def step(t, _):
      base = pl.multiple_of(e0 + t * CHUNK, CHUNK)
      pltpu.sync_copy(src_hbm.at[pl.ds(base, CHUNK)], src_v)
      pltpu.sync_copy(dst_hbm.at[pl.ds(base, CHUNK)], dst_v)
      pltpu.async_copy(y_hbm.at[src_v], rows_v, sem).wait()
      pltpu.sync_copy(rows_v, agg_sh.at[dst_v], add=True)
      if with_deg:
        pltpu.sync_copy(ones_v, deg_sh.at[dst_v], add=True)
      return 0
    lax.fori_loop(0, n_chunks, step, 0)

    plsc.subcore_barrier()

    pltpu.sync_copy(agg_sh.at[pl.ds(r0, rows_per_tile)],
                    agg_out.at[cid, pl.ds(r0, rows_per_tile)])
    if with_deg:
      pltpu.sync_copy(deg_sh.at[pl.ds(r0, rows_per_tile)],
                      deg_out.at[cid, pl.ds(r0, rows_per_tile)])

  return pl.kernel(body, out_type=out_type, mesh=mesh, scratch_types=scratch)


def _dot(a, b):
  return jnp.dot(a, b, preferred_element_type=jnp.float32,
                 precision=lax.Precision.HIGHEST)


def _tc_pre(x_ref, wn1_ref, ws1_ref, b1_ref, y1_ref, s1_ref):
  x = x_ref[...]
  y1_ref[...] = _dot(x, wn1_ref[...])
  s1_ref[...] = _dot(x, ws1_ref[...]) + b1_ref[...]


def _tc_mid(s1_ref, agg_ref, deg_ref, wn2_ref, wp_ref, h_ref, y2_ref):
  agg = agg_ref[0] + agg_ref[1]
  deg = deg_ref[0, :, 0:1] + deg_ref[1, :, 0:1]
  rdeg = 1.0 / jnp.maximum(deg, 1.0)
  h = jnp.maximum(s1_ref[...] + agg * rdeg, 0.0)
  h_ref[...] = h
  y2_ref[...] = _dot(_dot(h, wn2_ref[...]), wp_ref[...])


def _tc_post(h_ref, agg_ref, deg_ref, ws2_ref, wp_ref, b2p_ref, out_ref):
  agg = agg_ref[0] + agg_ref[1]
  deg = deg_ref[0, :, 0:1] + deg_ref[1, :, 0:1]
  rdeg = 1.0 / jnp.maximum(deg, 1.0)
  out_ref[...] = (_dot(_dot(h_ref[...], ws2_ref[...]), wp_ref[...])
                  + agg * rdeg + b2p_ref[...])


def kernel(x, edge_index, W_self1, W_neigh1, b1, W_self2, W_neigh2, b2,
           W_proj, b_proj):
  n, d = x.shape
  e = edge_index.shape[1]

  n_pad = ((n + 1 + NS - 1) // NS) * NS
  e_pad = ((e + NW * CHUNK - 1) // (NW * CHUNK)) * (NW * CHUNK)

  src = edge_index[0].astype(jnp.int32)
  dst = edge_index[1].astype(jnp.int32)
  pad = e_pad - e
  if pad:
    src = jnp.concatenate([src, jnp.zeros((pad,), jnp.int32)])
    dst = jnp.concatenate([dst, jnp.full((pad,), n, jnp.int32)])

  blk = 2000
  grid = (n // blk,)
  row_spec = pl.BlockSpec((blk, d), lambda i: (i, 0))
  w_spec = pl.BlockSpec((d, d), lambda i: (0, 0))
  b_spec = pl.BlockSpec((1, d), lambda i: (0, 0))
  agg_spec = pl.BlockSpec((NC, blk, d), lambda i: (0, i, 0))
  deg_spec = pl.BlockSpec((NC, blk, LANES), lambda i: (0, i, 0))
  row_out = jax.ShapeDtypeStruct((n, d), jnp.float32)

  y1, s1 = pl.pallas_call(
      _tc_pre, grid=grid,
      in_specs=[row_spec, w_spec, w_spec, b_spec],
      out_specs=[row_spec, row_spec],
      out_shape=[row_out, row_out],
  )(x, W_neigh1, W_self1, b1.reshape(1, d))

  sc_agg_deg = _sc_aggregate(n, n_pad, e_pad, d, with_deg=True)
  agg1, deg = sc_agg_deg(y1, src, dst)

  h, y2 = pl.pallas_call(
      _tc_mid, grid=grid,
      in_specs=[row_spec, agg_spec, deg_spec, w_spec, w_spec],
      out_specs=[row_spec, row_spec],
      out_shape=[row_out, row_out],
  )(s1, agg1, deg, W_neigh2, W_proj)

  sc_agg = _sc_aggregate(n, n_pad, e_pad, d, with_deg=False)
  (agg2,) = sc_agg(y2, src, dst)

  b2p = (b2 @ W_proj + b_proj).reshape(1, d)

  out = pl.pallas_call(
      _tc_post, grid=grid,
      in_specs=[row_spec, agg_spec, deg_spec, w_spec, w_spec, b_spec],
      out_specs=row_spec,
      out_shape=row_out,
  )(h, agg2, deg, W_self2, W_proj, b2p)

  return out


# speedup vs baseline: 3.8601x; 3.8601x over previous
import jax
import jax.numpy as jnp
from jax import lax
from jax.experimental import pallas as pl
from jax.experimental.pallas import tpu as pltpu
from jax.experimental.pallas import tpu_sc as plsc

NC = 2
NS = 16
NW = NC * NS
CHUNK = 128
LANES = 16


def _mesh():
  return plsc.VectorSubcoreMesh(
      core_axis_name="c", subcore_axis_name="s", num_cores=NC,
      num_subcores=NS)


def _sc_aggregate(n_pad, e_pad, d):
  e_per_tile = e_pad // NW
  n_chunks = e_per_tile // CHUNK
  rows_per_tile = n_pad // NS

  def body(y_hbm, src_hbm, dst_hbm, agg_out, src_v, dst_v, rows_v, agg_sh,
           sem):
    cid = lax.axis_index("c")
    sid = lax.axis_index("s")

    def zrow(i, _):
      def zcol(j, _):
        rows_v[i, pl.ds(j * LANES, LANES)] = jnp.zeros((LANES,), jnp.float32)
        return 0
      return lax.fori_loop(0, d // LANES, zcol, 0)
    lax.fori_loop(0, CHUNK, zrow, 0)

    r0 = sid * rows_per_tile
    n_zcopies = rows_per_tile // CHUNK
    for k in range(n_zcopies):
      pltpu.sync_copy(rows_v, agg_sh.at[pl.ds(r0 + k * CHUNK, CHUNK)])
    rem = rows_per_tile - n_zcopies * CHUNK
    if rem:
      pltpu.sync_copy(rows_v.at[pl.ds(0, rem)],
                      agg_sh.at[pl.ds(r0 + n_zcopies * CHUNK, rem)])

    plsc.subcore_barrier()

    wid = cid * NS + sid
    e0 = pl.multiple_of(wid * e_per_tile, CHUNK)

    def step(t, _):
      base = pl.multiple_of(e0 + t * CHUNK, CHUNK)
      pltpu.sync_copy(src_hbm.at[pl.ds(base, CHUNK)], src_v)
      pltpu.sync_copy(dst_hbm.at[pl.ds(base, CHUNK)], dst_v)
      pltpu.async_copy(y_hbm.at[src_v], rows_v, sem).wait()
      pltpu.sync_copy(rows_v, agg_sh.at[dst_v], add=True)
      return 0
    lax.fori_loop(0, n_chunks, step, 0)

    plsc.subcore_barrier()

    pltpu.sync_copy(agg_sh.at[pl.ds(r0, rows_per_tile)],
                    agg_out.at[cid, pl.ds(r0, rows_per_tile)])

  return pl.kernel(
      body,
      out_type=[jax.ShapeDtypeStruct((NC, n_pad, d), jnp.float32)],
      mesh=_mesh(),
      scratch_types=[
          pltpu.VMEM((CHUNK,), jnp.int32),
          pltpu.VMEM((CHUNK,), jnp.int32),
          pltpu.VMEM((CHUNK, d), jnp.float32),
          pltpu.VMEM_SHARED((n_pad, d), jnp.float32),
          pltpu.SemaphoreType.DMA,
      ])


def _sc_degree(n_pad, e_pad, d):
  e_per_tile = e_pad // NW
  n_chunks = e_per_tile // CHUNK
  rows_per_tile = n_pad // NS

  def body(dst_hbm, deg_out, dst_v, rows_v, deg_sh):
    cid = lax.axis_index("c")
    sid = lax.axis_index("s")

    def fill(val):
      def frow(i, _):
        def fcol(j, _):
          rows_v[i, pl.ds(j * LANES, LANES)] = jnp.full(
              (LANES,), val, jnp.float32)
          return 0
        return lax.fori_loop(0, d // LANES, fcol, 0)
      lax.fori_loop(0, CHUNK, frow, 0)

    fill(0.0)
    r0 = sid * rows_per_tile
    n_zcopies = rows_per_tile // CHUNK
    for k in range(n_zcopies):
      pltpu.sync_copy(rows_v, deg_sh.at[pl.ds(r0 + k * CHUNK, CHUNK)])
    rem = rows_per_tile - n_zcopies * CHUNK
    if rem:
      pltpu.sync_copy(rows_v.at[pl.ds(0, rem)],
                      deg_sh.at[pl.ds(r0 + n_zcopies * CHUNK, rem)])

    plsc.subcore_barrier()

    fill(1.0)
    wid = cid * NS + sid
    e0 = pl.multiple_of(wid * e_per_tile, CHUNK)

    def step(t, _):
      base = pl.multiple_of(e0 + t * CHUNK, CHUNK)
      pltpu.sync_copy(dst_hbm.at[pl.ds(base, CHUNK)], dst_v)
      pltpu.sync_copy(rows_v, deg_sh.at[dst_v], add=True)
      return 0
    lax.fori_loop(0, n_chunks, step, 0)

    plsc.subcore_barrier()

    pltpu.sync_copy(deg_sh.at[pl.ds(r0, rows_per_tile)],
                    deg_out.at[cid, pl.ds(r0, rows_per_tile)])

  return pl.kernel(
      body,
      out_type=[jax.ShapeDtypeStruct((NC, n_pad, d), jnp.float32)],
      mesh=_mesh(),
      scratch_types=[
          pltpu.VMEM((CHUNK,), jnp.int32),
          pltpu.VMEM((CHUNK, d), jnp.float32),
          pltpu.VMEM_SHARED((n_pad, d), jnp.float32),
      ])


def _dot(a, b):
  return jnp.dot(a, b, preferred_element_type=jnp.float32,
                 precision=lax.Precision.HIGHEST)


def _tc_pre(x_ref, wn1_ref, ws1_ref, b1_ref, y1_ref, s1_ref):
  x = x_ref[...]
  y1_ref[...] = _dot(x, wn1_ref[...])
  s1_ref[...] = _dot(x, ws1_ref[...]) + b1_ref[...]


def _tc_mid(s1_ref, agg_ref, deg_ref, wn2_ref, wp_ref, h_ref, y2_ref):
  agg = agg_ref[0] + agg_ref[1]
  rdeg = 1.0 / jnp.maximum(deg_ref[0] + deg_ref[1], 1.0)
  h = jnp.maximum(s1_ref[...] + agg * rdeg, 0.0)
  h_ref[...] = h
  y2_ref[...] = _dot(_dot(h, wn2_ref[...]), wp_ref[...])


def _tc_post(h_ref, agg_ref, deg_ref, ws2_ref, wp_ref, b2p_ref, out_ref):
  agg = agg_ref[0] + agg_ref[1]
  rdeg = 1.0 / jnp.maximum(deg_ref[0] + deg_ref[1], 1.0)
  out_ref[...] = (_dot(_dot(h_ref[...], ws2_ref[...]), wp_ref[...])
                  + agg * rdeg + b2p_ref[...])


def kernel(x, edge_index, W_self1, W_neigh1, b1, W_self2, W_neigh2, b2,
           W_proj, b_proj):
  n, d = x.shape
  e = edge_index.shape[1]

  n_pad = ((n + 1 + NS * 8 - 1) // (NS * 8)) * (NS * 8)
  e_pad = ((e + NW * CHUNK - 1) // (NW * CHUNK)) * (NW * CHUNK)

  src = edge_index[0].astype(jnp.int32)
  dst = edge_index[1].astype(jnp.int32)
  pad = e_pad - e
  if pad:
    src = jnp.concatenate([src, jnp.zeros((pad,), jnp.int32)])
    dst = jnp.concatenate([dst, jnp.full((pad,), n, jnp.int32)])

  blk = 2000
  grid = (n // blk,)
  row_spec = pl.BlockSpec((blk, d), lambda i: (i, 0))
  w_spec = pl.BlockSpec((d, d), lambda i: (0, 0))
  b_spec = pl.BlockSpec((1, d), lambda i: (0, 0))
  agg_spec = pl.BlockSpec((NC, blk, d), lambda i: (0, i, 0))
  row_out = jax.ShapeDtypeStruct((n, d), jnp.float32)

  y1, s1 = pl.pallas_call(
      _tc_pre, grid=grid,
      in_specs=[row_spec, w_spec, w_spec, b_spec],
      out_specs=[row_spec, row_spec],
      out_shape=[row_out, row_out],
  )(x, W_neigh1, W_self1, b1.reshape(1, d))

  sc_agg = _sc_aggregate(n_pad, e_pad, d)
  sc_deg = _sc_degree(n_pad, e_pad, d)
  (deg,) = sc_deg(dst)
  (agg1,) = sc_agg(y1, src, dst)

  h, y2 = pl.pallas_call(
      _tc_mid, grid=grid,
      in_specs=[row_spec, agg_spec, agg_spec, w_spec, w_spec],
      out_specs=[row_spec, row_spec],
      out_shape=[row_out, row_out],
  )(s1, agg1, deg, W_neigh2, W_proj)

  (agg2,) = sc_agg(y2, src, dst)

  b2p = (b2 @ W_proj + b_proj).reshape(1, d)

  out = pl.pallas_call(
      _tc_post, grid=grid,
      in_specs=[row_spec, agg_spec, agg_spec, w_spec, w_spec, b_spec],
      out_specs=row_spec,
      out_shape=row_out,
  )(h, agg2, deg, W_self2, W_proj, b2p)

  return out
